# Initial kernel scaffold; baseline (speedup 1.0000x reference)
import jax
import jax.numpy as jnp
from jax.experimental import pallas as pl


def kernel(embedding_user, embedding_item, graph_vals, graph_rows, graph_cols):
    raise NotImplementedError("write your pallas kernel here")


# speedup vs baseline: 7.8838x; 7.8838x over previous
import functools

import jax
import jax.numpy as jnp
from jax import lax
from jax.experimental import pallas as pl
from jax.experimental.pallas import tpu as pltpu
from jax.experimental.pallas import tpu_sc as plsc

_D = 128
_LANES = 16
_CH = 128
_NSUB = 16
_KP1 = 31


def _sc_edge_pass(src, gat_idx, dst_idx, vals16, zeros, n_dst, split_dst):
    e_total = dst_idx.shape[-1]
    n_acc = zeros.shape[0]
    epw = e_total // (_NSUB if split_dst else 2 * _NSUB)
    nch = epw // _CH
    stripe = n_acc // _NSUB
    mesh = plsc.VectorSubcoreMesh(core_axis_name="c", subcore_axis_name="s")

    @functools.partial(
        pl.kernel,
        mesh=mesh,
        out_type=jax.ShapeDtypeStruct(
            (n_dst if split_dst else 2 * n_dst, _D), jnp.float32),
        scratch_types=[
            pltpu.VMEM_SHARED((n_acc + (8 if split_dst else 0), _D),
                              jnp.float32),
            pltpu.VMEM((_CH,), jnp.int32),
            pltpu.VMEM((_CH,), jnp.int32),
            pltpu.VMEM((_CH, _LANES), jnp.float32),
            pltpu.VMEM((_CH, _D), jnp.float32),
        ],
    )
    def pass_kernel(src_hbm, idx_hbm, dsti_hbm, vals_hbm, zeros_hbm, out_hbm,
                    acc, idx_v, dsti_v, vals_v, rows_v):
        c = lax.axis_index("c")
        s = lax.axis_index("s")
        pltpu.sync_copy(zeros_hbm.at[pl.ds(s * stripe, stripe)],
                        acc.at[pl.ds(s * stripe, stripe)])
        plsc.subcore_barrier()
        base0 = (s if split_dst else c * _NSUB + s) * epw

        @pl.loop(0, nch)
        def _(t):
            b = base0 + t * _CH
            pltpu.sync_copy(idx_hbm.at[pl.ds(b, _CH)], idx_v)
            if split_dst:
                pltpu.sync_copy(dsti_hbm.at[c, pl.ds(b, _CH)], dsti_v)
            else:
                pltpu.sync_copy(dsti_hbm.at[pl.ds(b, _CH)], dsti_v)
            pltpu.sync_copy(vals_hbm.at[pl.ds(b, _CH), :], vals_v)
            pltpu.sync_copy(src_hbm.at[idx_v], rows_v)

            @pl.loop(0, _CH)
            def _(e):
                vv = vals_v[e, :]
                for j in range(_D // _LANES):
                    sl = pl.ds(j * _LANES, _LANES)
                    rows_v[e, sl] = rows_v[e, sl] * vv

            pltpu.sync_copy(rows_v, acc.at[dsti_v], add=True)

        plsc.subcore_barrier()
        obase = (c * n_acc if split_dst else c * n_dst) + s * stripe
        pltpu.sync_copy(acc.at[pl.ds(s * stripe, stripe)],
                        out_hbm.at[pl.ds(obase, stripe)])

    return pass_kernel(src, gat_idx, dst_idx, vals16, zeros)


def _tc_combine_normalize(e0, p1a, p1b, p2a, p2b):
    n, d = e0.shape

    def body(r0, ra, rb, rc, rd, o_ref):
        m = (r0[...] + ra[...] + rb[...] + rc[...] + rd[...]) * (1.0 / 3.0)
        nrm = jnp.sqrt(jnp.sum(m * m, axis=1, keepdims=True))
        o_ref[...] = m / jnp.maximum(nrm, 1e-12)

    return pl.pallas_call(
        body,
        out_shape=jax.ShapeDtypeStruct((n, d), jnp.float32),
    )(e0, p1a, p1b, p2a, p2b)


def _tc_sim_topk(emb):
    n, d = emb.shape
    br = 256

    def body(rows_ref, emb_ref, o_ref):
        sim = lax.dot_general(rows_ref[...], emb_ref[...],
                              (((1,), (1,)), ((), ())),
                              preferred_element_type=jnp.float32)
        s = jnp.maximum(sim, 0.0)
        bits = lax.bitcast_convert_type(s, jnp.int32)
        lo0 = jnp.zeros((br, 1), jnp.int32)
        hi0 = jnp.full((br, 1), 0x40000000, jnp.int32)

        def step(_, lh):
            lo, hi = lh
            mid = (lo + hi) >> 1
            cnt = jnp.sum((bits >= mid).astype(jnp.int32), axis=1,
                          keepdims=True)
            ge = cnt >= _KP1
            return jnp.where(ge, mid, lo), jnp.where(ge, hi, mid)

        lo, _ = lax.fori_loop(0, 30, step, (lo0, hi0))
        thr = jnp.maximum(lo, 1)
        o_ref[...] = jnp.where(bits >= thr, s, 0.0)

    return pl.pallas_call(
        body,
        grid=(n // br,),
        in_specs=[
            pl.BlockSpec((br, d), lambda i: (i, 0)),
            pl.BlockSpec((n, d), lambda i: (0, 0)),
        ],
        out_specs=pl.BlockSpec((br, n), lambda i: (i, 0)),
        out_shape=jax.ShapeDtypeStruct((n, n), jnp.float32),
    )(emb, emb)


def kernel(embedding_user, embedding_item, graph_vals, graph_rows, graph_cols):
    nu = embedding_user.shape[0]
    ni = embedding_item.shape[0]
    e = graph_vals.shape[0] // 2

    rows_u = graph_rows[:e]
    cols_i = graph_cols[:e] - nu
    rows_i = graph_rows[e:] - nu
    cols_u = graph_cols[e:]
    va16 = jnp.broadcast_to(graph_vals[:e, None], (e, _LANES))
    vb16 = jnp.broadcast_to(graph_vals[e:, None], (e, _LANES))
    zeros_u = jnp.zeros((nu // 2, _D), jnp.float32)
    zeros_i = jnp.zeros((ni, _D), jnp.float32)

    nh = nu // 2
    ru2 = jnp.stack([jnp.where((rows_u >= c * nh) & (rows_u < (c + 1) * nh),
                               rows_u - c * nh, nh) for c in range(2)])

    p1 = _sc_edge_pass(embedding_user, cols_u, rows_i, vb16, zeros_i, ni,
                       split_dst=False)
    e1u = _sc_edge_pass(embedding_item, cols_i, ru2, va16, zeros_u, nu,
                        split_dst=True)
    p3 = _sc_edge_pass(e1u, cols_u, rows_i, vb16, zeros_i, ni,
                       split_dst=False)

    emb = _tc_combine_normalize(embedding_item, p1[:ni], p1[ni:],
                                p3[:ni], p3[ni:])
    return _tc_sim_topk(emb)


# speedup vs baseline: 11.0363x; 1.3999x over previous
import functools

import jax
import jax.numpy as jnp
from jax import lax
from jax.experimental import pallas as pl
from jax.experimental.pallas import tpu as pltpu
from jax.experimental.pallas import tpu_sc as plsc

_D = 128
_LANES = 16
_CH = 128
_NSUB = 16
_KP1 = 31


def _sc_edge_pass(src, gat_idx, dst_idx, vals16, zeros, n_dst, split_dst):
    e_total = dst_idx.shape[-1]
    n_acc = zeros.shape[0]
    ch = _CH // 2 if split_dst else _CH
    epw = e_total // (_NSUB if split_dst else 2 * _NSUB)
    nch = epw // ch
    stripe = n_acc // _NSUB
    mesh = plsc.VectorSubcoreMesh(core_axis_name="c", subcore_axis_name="s")

    @functools.partial(
        pl.kernel,
        mesh=mesh,
        out_type=jax.ShapeDtypeStruct(
            (n_dst if split_dst else 2 * n_dst, _D), jnp.float32),
        scratch_types=[
            pltpu.VMEM_SHARED((n_acc + (8 if split_dst else 0), _D),
                              jnp.float32),
            pltpu.VMEM((ch,), jnp.int32),
            pltpu.VMEM((ch,), jnp.int32),
            pltpu.VMEM((ch,), jnp.int32),
            pltpu.VMEM((ch,), jnp.int32),
            pltpu.VMEM((ch, _LANES), jnp.float32),
            pltpu.VMEM((ch, _LANES), jnp.float32),
            pltpu.VMEM((ch, _D), jnp.float32),
            pltpu.VMEM((ch, _D), jnp.float32),
            pltpu.SemaphoreType.DMA,
            pltpu.SemaphoreType.DMA,
            pltpu.SemaphoreType.DMA,
            pltpu.SemaphoreType.DMA,
        ],
    )
    def pass_kernel(src_hbm, idx_hbm, dsti_hbm, vals_hbm, zeros_hbm, out_hbm,
                    acc, idx0, idx1, dst0, dst1, val0, val1, row0, row1,
                    sin0, sin1, sg0, sg1):
        c = lax.axis_index("c")
        s = lax.axis_index("s")
        idx_b, dst_b, val_b, row_b = (idx0, idx1), (dst0, dst1), \
            (val0, val1), (row0, row1)
        sin_b, sg_b = (sin0, sin1), (sg0, sg1)
        pltpu.sync_copy(zeros_hbm.at[pl.ds(s * stripe, stripe)],
                        acc.at[pl.ds(s * stripe, stripe)])
        plsc.subcore_barrier()
        base0 = (s if split_dst else c * _NSUB + s) * epw

        def fire_inputs(t, bb):
            b = base0 + t * ch
            pltpu.async_copy(idx_hbm.at[pl.ds(b, ch)], idx_b[bb], sin_b[bb])
            if split_dst:
                pltpu.async_copy(dsti_hbm.at[c, pl.ds(b, ch)], dst_b[bb],
                                 sin_b[bb])
            else:
                pltpu.async_copy(dsti_hbm.at[pl.ds(b, ch)], dst_b[bb],
                                 sin_b[bb])
            pltpu.async_copy(vals_hbm.at[pl.ds(b, ch), :], val_b[bb],
                             sin_b[bb])

        def wait_inputs(bb):
            pltpu.make_async_copy(idx_hbm.at[pl.ds(0, ch)], idx_b[bb],
                                  sin_b[bb]).wait()
            if split_dst:
                pltpu.make_async_copy(dsti_hbm.at[0, pl.ds(0, ch)],
                                      dst_b[bb], sin_b[bb]).wait()
            else:
                pltpu.make_async_copy(dsti_hbm.at[pl.ds(0, ch)], dst_b[bb],
                                      sin_b[bb]).wait()
            pltpu.make_async_copy(vals_hbm.at[pl.ds(0, ch), :], val_b[bb],
                                  sin_b[bb]).wait()

        def fire_gather(bb):
            pltpu.async_copy(src_hbm.at[idx_b[bb]], row_b[bb], sg_b[bb])

        def wait_gather(bb):
            pltpu.make_async_copy(src_hbm.at[idx_b[bb]], row_b[bb],
                                  sg_b[bb]).wait()

        def scale_scatter(bb):
            rv, vv_ref = row_b[bb], val_b[bb]

            @pl.loop(0, ch)
            def _(e):
                vv = vv_ref[e, :]
                for j in range(_D // _LANES):
                    sl = pl.ds(j * _LANES, _LANES)
                    rv[e, sl] = rv[e, sl] * vv

            pltpu.sync_copy(rv, acc.at[dst_b[bb]], add=True)

        fire_inputs(0, 0)
        wait_inputs(0)
        fire_gather(0)
        fire_inputs(1, 1)

        @pl.loop(0, nch // 2 - 1)
        def _(u):
            for bb in range(2):
                t = 2 * u + bb
                wait_gather(bb)
                wait_inputs(1 - bb)
                fire_gather(1 - bb)
                scale_scatter(bb)
                fire_inputs(t + 2, bb)

        wait_gather(0)
        wait_inputs(1)
        fire_gather(1)
        scale_scatter(0)
        wait_gather(1)
        scale_scatter(1)

        plsc.subcore_barrier()
        obase = (c * n_acc if split_dst else c * n_dst) + s * stripe
        pltpu.sync_copy(acc.at[pl.ds(s * stripe, stripe)],
                        out_hbm.at[pl.ds(obase, stripe)])

    return pass_kernel(src, gat_idx, dst_idx, vals16, zeros)


def _tc_combine_normalize(e0, p1a, p1b, p2a, p2b):
    n, d = e0.shape

    def body(r0, ra, rb, rc, rd, o_ref):
        m = (r0[...] + ra[...] + rb[...] + rc[...] + rd[...]) * (1.0 / 3.0)
        nrm = jnp.sqrt(jnp.sum(m * m, axis=1, keepdims=True))
        o_ref[...] = m / jnp.maximum(nrm, 1e-12)

    return pl.pallas_call(
        body,
        out_shape=jax.ShapeDtypeStruct((n, d), jnp.float32),
    )(e0, p1a, p1b, p2a, p2b)


def _tc_sim_topk(emb):
    n, d = emb.shape
    br = 256

    def body(rows_ref, emb_ref, o_ref):
        sim = lax.dot_general(rows_ref[...], emb_ref[...],
                              (((1,), (1,)), ((), ())),
                              preferred_element_type=jnp.float32)
        s = jnp.maximum(sim, 0.0)
        bits = lax.bitcast_convert_type(s, jnp.int32)
        lo0 = jnp.zeros((br, 1), jnp.int32)
        hi0 = jnp.full((br, 1), 0x40000000, jnp.int32)

        def step(_, lh):
            lo, hi = lh
            mid = (lo + hi) >> 1
            cnt = jnp.sum((bits >= mid).astype(jnp.int32), axis=1,
                          keepdims=True)
            ge = cnt >= _KP1
            return jnp.where(ge, mid, lo), jnp.where(ge, hi, mid)

        lo, _ = lax.fori_loop(0, 30, step, (lo0, hi0))
        thr = jnp.maximum(lo, 1)
        o_ref[...] = jnp.where(bits >= thr, s, 0.0)

    return pl.pallas_call(
        body,
        grid=(n // br,),
        in_specs=[
            pl.BlockSpec((br, d), lambda i: (i, 0)),
            pl.BlockSpec((n, d), lambda i: (0, 0)),
        ],
        out_specs=pl.BlockSpec((br, n), lambda i: (i, 0)),
        out_shape=jax.ShapeDtypeStruct((n, n), jnp.float32),
    )(emb, emb)


def kernel(embedding_user, embedding_item, graph_vals, graph_rows, graph_cols):
    nu = embedding_user.shape[0]
    ni = embedding_item.shape[0]
    e = graph_vals.shape[0] // 2

    rows_u = graph_rows[:e]
    cols_i = graph_cols[:e] - nu
    rows_i = graph_rows[e:] - nu
    cols_u = graph_cols[e:]
    va16 = jnp.broadcast_to(graph_vals[:e, None], (e, _LANES))
    vb16 = jnp.broadcast_to(graph_vals[e:, None], (e, _LANES))
    zeros_u = jnp.zeros((nu // 2, _D), jnp.float32)
    zeros_i = jnp.zeros((ni, _D), jnp.float32)

    nh = nu // 2
    ru2 = jnp.stack([jnp.where((rows_u >= c * nh) & (rows_u < (c + 1) * nh),
                               rows_u - c * nh, nh) for c in range(2)])

    p1 = _sc_edge_pass(embedding_user, cols_u, rows_i, vb16, zeros_i, ni,
                       split_dst=False)
    e1u = _sc_edge_pass(embedding_item, cols_i, ru2, va16, zeros_u, nu,
                        split_dst=True)
    p3 = _sc_edge_pass(e1u, cols_u, rows_i, vb16, zeros_i, ni,
                       split_dst=False)

    emb = _tc_combine_normalize(embedding_item, p1[:ni], p1[ni:],
                                p3[:ni], p3[ni:])
    return _tc_sim_topk(emb)


# speedup vs baseline: 13.3105x; 1.2061x over previous
import functools

import jax
import jax.numpy as jnp
from jax import lax
from jax.experimental import pallas as pl
from jax.experimental.pallas import tpu as pltpu
from jax.experimental.pallas import tpu_sc as plsc

_D = 128
_LANES = 16
_CH = 128
_NSUB = 16
_KP1 = 31


def _sc_edge_pass(src, gat_idx, dst_idx, vals, zeros, n_dst, split_dst,
                  src_in_spmem=False):
    e_total = dst_idx.shape[-1]
    n_src = src.shape[0]
    n_acc = zeros.shape[0]
    ch = _CH // 2 if split_dst else _CH
    epw = e_total // (_NSUB if split_dst else 2 * _NSUB)
    nch = epw // ch
    stripe = n_acc // _NSUB
    mesh = plsc.VectorSubcoreMesh(core_axis_name="c", subcore_axis_name="s")

    @functools.partial(
        pl.kernel,
        mesh=mesh,
        out_type=jax.ShapeDtypeStruct(
            (n_dst if split_dst else 2 * n_dst, _D), jnp.float32),
        scratch_types=[
            pltpu.VMEM_SHARED((n_acc + (8 if split_dst else 0), _D),
                              jnp.float32),
            pltpu.VMEM((ch,), jnp.int32),
            pltpu.VMEM((ch,), jnp.int32),
            pltpu.VMEM((ch,), jnp.int32),
            pltpu.VMEM((ch,), jnp.int32),
            pltpu.VMEM((ch,), jnp.float32),
            pltpu.VMEM((ch,), jnp.float32),
            pltpu.VMEM((ch, _D), jnp.float32),
            pltpu.VMEM((ch, _D), jnp.float32),
            pltpu.SemaphoreType.DMA,
            pltpu.SemaphoreType.DMA,
            pltpu.SemaphoreType.DMA,
            pltpu.SemaphoreType.DMA,
        ] + ([pltpu.VMEM_SHARED((n_src, _D), jnp.float32)]
             if src_in_spmem else []),
    )
    def pass_kernel(src_hbm, idx_hbm, dsti_hbm, vals_hbm, zeros_hbm, out_hbm,
                    acc, idx0, idx1, dst0, dst1, val0, val1, row0, row1,
                    sin0, sin1, sg0, sg1, *table_sc):
        c = lax.axis_index("c")
        s = lax.axis_index("s")
        gsrc = table_sc[0] if src_in_spmem else src_hbm
        idx_b, dst_b, val_b, row_b = (idx0, idx1), (dst0, dst1), \
            (val0, val1), (row0, row1)
        sin_b, sg_b = (sin0, sin1), (sg0, sg1)
        pltpu.sync_copy(zeros_hbm.at[pl.ds(s * stripe, stripe)],
                        acc.at[pl.ds(s * stripe, stripe)])
        if src_in_spmem:
            tst = n_src // _NSUB
            pltpu.sync_copy(src_hbm.at[pl.ds(s * tst, tst)],
                            table_sc[0].at[pl.ds(s * tst, tst)])
        plsc.subcore_barrier()
        base0 = (s if split_dst else c * _NSUB + s) * epw

        def fire_inputs(t, bb):
            b = base0 + t * ch
            pltpu.async_copy(idx_hbm.at[pl.ds(b, ch)], idx_b[bb], sin_b[bb])
            if split_dst:
                pltpu.async_copy(dsti_hbm.at[c, pl.ds(b, ch)], dst_b[bb],
                                 sin_b[bb])
            else:
                pltpu.async_copy(dsti_hbm.at[pl.ds(b, ch)], dst_b[bb],
                                 sin_b[bb])
            pltpu.async_copy(vals_hbm.at[pl.ds(b, ch)], val_b[bb],
                             sin_b[bb])

        def wait_inputs(bb):
            pltpu.make_async_copy(idx_hbm.at[pl.ds(0, ch)], idx_b[bb],
                                  sin_b[bb]).wait()
            if split_dst:
                pltpu.make_async_copy(dsti_hbm.at[0, pl.ds(0, ch)],
                                      dst_b[bb], sin_b[bb]).wait()
            else:
                pltpu.make_async_copy(dsti_hbm.at[pl.ds(0, ch)], dst_b[bb],
                                      sin_b[bb]).wait()
            pltpu.make_async_copy(vals_hbm.at[pl.ds(0, ch)], val_b[bb],
                                  sin_b[bb]).wait()

        def fire_gather(bb):
            pltpu.async_copy(gsrc.at[idx_b[bb]], row_b[bb], sg_b[bb])

        def wait_gather(bb):
            pltpu.make_async_copy(gsrc.at[idx_b[bb]], row_b[bb],
                                  sg_b[bb]).wait()

        def scale_scatter(bb):
            rv, vv_ref = row_b[bb], val_b[bb]

            @pl.loop(0, ch // _LANES)
            def _(g):
                vv16 = vv_ref[pl.ds(g * _LANES, _LANES)]
                if split_dst:
                    dv16 = dst_b[bb][pl.ds(g * _LANES, _LANES)]
                for k in range(_LANES):
                    e = g * _LANES + k

                    def scale_row():
                        vvk = jnp.full((_LANES,), vv16[k], jnp.float32)
                        for j in range(_D // _LANES):
                            sl = pl.ds(j * _LANES, _LANES)
                            rv[e, sl] = rv[e, sl] * vvk

                    if split_dst:
                        pl.when(dv16[k] < n_acc)(scale_row)
                    else:
                        scale_row()

            pltpu.sync_copy(rv, acc.at[dst_b[bb]], add=True)

        fire_inputs(0, 0)
        wait_inputs(0)
        fire_gather(0)
        fire_inputs(1, 1)

        @pl.loop(0, nch // 2 - 1)
        def _(u):
            for bb in range(2):
                t = 2 * u + bb
                wait_gather(bb)
                wait_inputs(1 - bb)
                fire_gather(1 - bb)
                scale_scatter(bb)
                fire_inputs(t + 2, bb)

        wait_gather(0)
        wait_inputs(1)
        fire_gather(1)
        scale_scatter(0)
        wait_gather(1)
        scale_scatter(1)

        plsc.subcore_barrier()
        obase = (c * n_acc if split_dst else c * n_dst) + s * stripe
        pltpu.sync_copy(acc.at[pl.ds(s * stripe, stripe)],
                        out_hbm.at[pl.ds(obase, stripe)])

    return pass_kernel(src, gat_idx, dst_idx, vals, zeros)


def _tc_combine_normalize(e0, p1a, p1b, p2a, p2b):
    n, d = e0.shape

    def body(r0, ra, rb, rc, rd, o_ref):
        m = (r0[...] + ra[...] + rb[...] + rc[...] + rd[...]) * (1.0 / 3.0)
        nrm = jnp.sqrt(jnp.sum(m * m, axis=1, keepdims=True))
        o_ref[...] = m / jnp.maximum(nrm, 1e-12)

    return pl.pallas_call(
        body,
        out_shape=jax.ShapeDtypeStruct((n, d), jnp.float32),
    )(e0, p1a, p1b, p2a, p2b)


def _tc_sim_topk(emb):
    n, d = emb.shape
    br = 256

    def body(rows_ref, emb_ref, o_ref):
        sim = lax.dot_general(rows_ref[...], emb_ref[...],
                              (((1,), (1,)), ((), ())),
                              preferred_element_type=jnp.float32)
        s = jnp.maximum(sim, 0.0)
        bits = lax.bitcast_convert_type(s, jnp.int32)
        lo0 = jnp.zeros((br, 1), jnp.int32)
        hi0 = jnp.full((br, 1), 0x40000000, jnp.int32)

        def step(_, lh):
            lo, hi = lh
            mid = (lo + hi) >> 1
            cnt = jnp.sum((bits >= mid).astype(jnp.int32), axis=1,
                          keepdims=True)
            ge = cnt >= _KP1
            return jnp.where(ge, mid, lo), jnp.where(ge, hi, mid)

        lo, _ = lax.fori_loop(0, 30, step, (lo0, hi0))
        thr = jnp.maximum(lo, 1)
        o_ref[...] = jnp.where(bits >= thr, s, 0.0)

    return pl.pallas_call(
        body,
        grid=(n // br,),
        in_specs=[
            pl.BlockSpec((br, d), lambda i: (i, 0)),
            pl.BlockSpec((n, d), lambda i: (0, 0)),
        ],
        out_specs=pl.BlockSpec((br, n), lambda i: (i, 0)),
        out_shape=jax.ShapeDtypeStruct((n, n), jnp.float32),
        compiler_params=pltpu.CompilerParams(
            dimension_semantics=("parallel",)),
    )(emb, emb)


def kernel(embedding_user, embedding_item, graph_vals, graph_rows, graph_cols):
    nu = embedding_user.shape[0]
    ni = embedding_item.shape[0]
    e = graph_vals.shape[0] // 2

    rows_u = graph_rows[:e]
    cols_i = graph_cols[:e] - nu
    rows_i = graph_rows[e:] - nu
    cols_u = graph_cols[e:]
    va = graph_vals[:e]
    vb = graph_vals[e:]
    zeros_u = jnp.zeros((nu // 2, _D), jnp.float32)
    zeros_i = jnp.zeros((ni, _D), jnp.float32)

    nh = nu // 2
    ru2 = jnp.stack([jnp.where((rows_u >= c * nh) & (rows_u < (c + 1) * nh),
                               rows_u - c * nh, nh) for c in range(2)])

    p1 = _sc_edge_pass(embedding_user, cols_u, rows_i, vb, zeros_i, ni,
                       split_dst=False)
    e1u = _sc_edge_pass(embedding_item, cols_i, ru2, va, zeros_u, nu,
                        split_dst=True, src_in_spmem=True)
    p3 = _sc_edge_pass(e1u, cols_u, rows_i, vb, zeros_i, ni,
                       split_dst=False)

    emb = _tc_combine_normalize(embedding_item, p1[:ni], p1[ni:],
                                p3[:ni], p3[ni:])
    return _tc_sim_topk(emb)
